# Initial kernel scaffold; baseline (speedup 1.0000x reference)
import jax
import jax.numpy as jnp
from jax.experimental import pallas as pl


def kernel(adj_input, r_index, r_val, t_index, ent_matrix, rel_matrix, time_matrix, ent_emb_r, ent_emb_t, rel_emb, time_emb, ak_e0, ak_e1, ak_t0, ak_t1):
    raise NotImplementedError("write your pallas kernel here")


# speedup vs baseline: 2.1225x; 2.1225x over previous
import functools
import jax
import jax.numpy as jnp
from jax import lax
from jax.experimental import pallas as pl
from jax.experimental.pallas import tpu as pltpu
from jax.experimental.pallas import tpu_sc as plsc

N = 10000
E = 320000
D = 128
H = 64
NP = 10240
EP = 327680
TRASH = 10000
RPT = 640
EPT = 20480
BC = 128
NCHK = EPT // BC

_mesh = plsc.VectorSubcoreMesh(core_axis_name="c", subcore_axis_name="s")
_params = pltpu.CompilerParams(use_tc_tiling_on_sc=False)
_f32 = jnp.float32
_i32 = jnp.int32


def _zero_vmem_f32(buf, nrows):
    z = jnp.zeros((16,), _f32)
    def body(r, _):
        for j in range(buf.shape[1] // 16):
            buf[r, pl.ds(16 * j, 16)] = z
        return 0
    lax.fori_loop(0, nrows, body, 0, unroll=False)


def _mean_body(seg_h, col_h, tab_h, outF_h,
               acc, cnt, segb, colb, rows, ones, fbuf, cbuf, sem):
    c = lax.axis_index("c")
    s = lax.axis_index("s")

    _zero_vmem_f32(fbuf, 128)
    z = jnp.zeros((16,), _f32)
    one = jnp.ones((16,), _f32)
    def initones(r, _):
        ones[pl.ds(16 * r, 16)] = one
        cbuf[pl.ds(16 * r, 16)] = z
        return 0
    lax.fori_loop(0, BC // 16, initones, 0, unroll=False)
    for b in range(RPT // 128):
        pltpu.sync_copy(fbuf, acc.at[pl.ds(s * RPT + 128 * b, 128)])
        pltpu.sync_copy(cbuf, cnt.at[pl.ds(s * RPT + 128 * b, 128)])
    plsc.subcore_barrier()

    ebase = s * EPT
    def chunk(k, _):
        base = ebase + k * BC
        pltpu.sync_copy(seg_h.at[pl.ds(base, BC)], segb.at[0])
        pltpu.sync_copy(col_h.at[pl.ds(base, BC)], colb.at[0])
        for j in range(BC // 16):
            v = colb[0, pl.ds(16 * j, 16)]
            colb[0, pl.ds(16 * j, 16)] = v * 2 + c
        pltpu.async_copy(tab_h.at[colb.at[0]], rows, sem).wait()
        pltpu.sync_copy(rows, acc.at[segb.at[0]], add=True)
        pltpu.sync_copy(ones, cnt.at[segb.at[0]], add=True)
        return 0
    lax.fori_loop(0, NCHK, chunk, 0, unroll=False)
    plsc.subcore_barrier()

    for b in range(RPT // 128):
        r0 = s * RPT + b * 128
        pltpu.sync_copy(acc.at[pl.ds(r0, 128)], fbuf)
        pltpu.sync_copy(cnt.at[pl.ds(r0, 128)], cbuf)
        def grp(g, _):
            cntv = cbuf[pl.ds(16 * g, 16)]
            recv = 1.0 / jnp.maximum(cntv, 1.0)
            for r in range(16):
                rec = recv[r]
                for j in range(4):
                    v = fbuf[16 * g + r, pl.ds(16 * j, 16)]
                    fbuf[16 * g + r, pl.ds(16 * j, 16)] = jnp.maximum(v * rec, 0.0)
            return 0
        lax.fori_loop(0, 8, grp, 0, unroll=False)
        pltpu.sync_copy(fbuf, outF_h.at[pl.ds(r0, 128), c])


_mean_call = functools.partial(
    pl.kernel, _mean_body,
    out_type=jax.ShapeDtypeStruct((NP, 2, H), _f32),
    mesh=_mesh,
    compiler_params=_params,
    scratch_types=[
        pltpu.VMEM_SHARED((NP, H), _f32),
        pltpu.VMEM_SHARED((NP,), _f32),
        pltpu.VMEM((1, BC), _i32),
        pltpu.VMEM((1, BC), _i32),
        pltpu.VMEM((BC, H), _f32),
        pltpu.VMEM((BC,), _f32),
        pltpu.VMEM((128, H), _f32),
        pltpu.VMEM((128,), _f32),
        pltpu.SemaphoreType.DMA,
    ],
)


def _segment_mean_sc(idx, table):
    seg = jnp.concatenate([idx[:, 0], jnp.full((EP - E,), TRASH, _i32)])
    col = jnp.concatenate([idx[:, 1], jnp.zeros((EP - E,), _i32)])
    tab2 = table.reshape(-1, 2, H).reshape(-1, H)
    return _mean_call()(seg, col, tab2)


def _kg_body(src_h, f1_h, f2_h, g1_h, g2_h, srcb, rows1, rows2, sem):
    c = lax.axis_index("c")
    s = lax.axis_index("s")

    def chunk(k, _):
        base = s * EPT + k * BC
        pltpu.sync_copy(src_h.at[pl.ds(base, BC)], srcb.at[0])
        for j in range(BC // 16):
            v = srcb[0, pl.ds(16 * j, 16)]
            srcb[0, pl.ds(16 * j, 16)] = v * 2 + c
        pltpu.async_copy(f1_h.at[srcb.at[0]], rows1, sem).wait()
        pltpu.async_copy(f2_h.at[srcb.at[0]], rows2, sem).wait()
        pltpu.sync_copy(rows1, g1_h.at[pl.ds(base, BC), c])
        pltpu.sync_copy(rows2, g2_h.at[pl.ds(base, BC), c])
        return 0
    lax.fori_loop(0, NCHK, chunk, 0, unroll=False)


_kg_call = functools.partial(
    pl.kernel, _kg_body,
    out_type=[jax.ShapeDtypeStruct((EP, 2, H), _f32),
              jax.ShapeDtypeStruct((EP, 2, H), _f32)],
    mesh=_mesh,
    compiler_params=_params,
    scratch_types=[
        pltpu.VMEM((1, BC), _i32),
        pltpu.VMEM((BC, H), _f32),
        pltpu.VMEM((BC, H), _f32),
        pltpu.SemaphoreType.DMA,
    ],
)


def _kw_body(dst_h, s1_h, s2_h, pl1_h, pl2_h,
             w1_h, w2_h, den1_h, den2_h,
             den_sh, dstb, sgb, plb, wb, sem):
    c = lax.axis_index("c")
    s = lax.axis_index("s")

    z = jnp.zeros((16,), _f32)
    def zrow(r, _):
        wb[pl.ds(16 * r, 16)] = z
        return 0
    lax.fori_loop(0, BC // 16, zrow, 0, unroll=False)
    for b in range(RPT // 128):
        pltpu.sync_copy(wb, den_sh.at[pl.ds(s * RPT + 128 * b, 128)])
    plsc.subcore_barrier()

    def phase1(s_h, pl_h, w_h):
        def chunk(k, _):
            base = s * EPT + k * BC
            pltpu.sync_copy(dst_h.at[pl.ds(base, BC)], dstb.at[0])
            pltpu.async_copy(s_h.at[dstb.at[0]], sgb, sem).wait()
            pltpu.sync_copy(pl_h.at[pl.ds(base, BC)], plb)
            def grp(g, _):
                dd = pl.ds(16 * g, 16)
                wb[dd] = jnp.exp(sgb[dd] + plb[dd])
                return 0
            lax.fori_loop(0, BC // 16, grp, 0, unroll=False)
            pltpu.sync_copy(wb, w_h.at[pl.ds(base, BC)])
            pltpu.sync_copy(wb, den_sh.at[dstb.at[0]], add=True)
            return 0
        lax.fori_loop(0, NCHK, chunk, 0, unroll=False)

    pl.when(c == 0)(lambda: phase1(s1_h, pl1_h, w1_h))
    pl.when(c == 1)(lambda: phase1(s2_h, pl2_h, w2_h))
    plsc.subcore_barrier()

    @pl.when(s == 0)
    def _():
        pl.when(c == 0)(lambda: pltpu.sync_copy(den_sh, den1_h))
        pl.when(c == 1)(lambda: pltpu.sync_copy(den_sh, den2_h))


_kw_call = functools.partial(
    pl.kernel, _kw_body,
    out_type=[jax.ShapeDtypeStruct((EP,), _f32),
              jax.ShapeDtypeStruct((EP,), _f32),
              jax.ShapeDtypeStruct((NP,), _f32),
              jax.ShapeDtypeStruct((NP,), _f32)],
    mesh=_mesh,
    compiler_params=_params,
    scratch_types=[
        pltpu.VMEM_SHARED((NP,), _f32),
        pltpu.VMEM((1, BC), _i32),
        pltpu.VMEM((BC,), _f32),
        pltpu.VMEM((BC,), _f32),
        pltpu.VMEM((BC,), _f32),
        pltpu.SemaphoreType.DMA,
    ],
)


def _ka_body(dst_h, src_h, f1_h, f2_h, rels_h, w1_h, w2_h, dn1_h, dn2_h,
             inv_h, den1_h, den2_h, g1_h, g2_h,
             acc1, acc2, dstb, srcb, rows1, rows2, relsb,
             wb1, wb2, db1, db2, ivb, dg1, dg2, fbuf, sem):
    c = lax.axis_index("c")
    s = lax.axis_index("s")

    _zero_vmem_f32(fbuf, 128)
    for b in range(RPT // 128):
        pltpu.sync_copy(fbuf, acc1.at[pl.ds(s * RPT + 128 * b, 128)])
        pltpu.sync_copy(fbuf, acc2.at[pl.ds(s * RPT + 128 * b, 128)])
    plsc.subcore_barrier()

    def chunk(k, _):
        base = s * EPT + k * BC
        pltpu.sync_copy(dst_h.at[pl.ds(base, BC)], dstb.at[0])
        pltpu.sync_copy(src_h.at[pl.ds(base, BC)], srcb.at[0])
        for j in range(BC // 16):
            v = srcb[0, pl.ds(16 * j, 16)]
            srcb[0, pl.ds(16 * j, 16)] = v * 2 + c
        pltpu.async_copy(f1_h.at[srcb.at[0]], rows1, sem).wait()
        pltpu.async_copy(f2_h.at[srcb.at[0]], rows2, sem).wait()
        pltpu.async_copy(den1_h.at[dstb.at[0]], dg1, sem).wait()
        pltpu.async_copy(den2_h.at[dstb.at[0]], dg2, sem).wait()
        pltpu.sync_copy(rels_h.at[pl.ds(base, BC), c], relsb)
        pltpu.sync_copy(w1_h.at[pl.ds(base, BC)], wb1)
        pltpu.sync_copy(w2_h.at[pl.ds(base, BC)], wb2)
        pltpu.sync_copy(dn1_h.at[pl.ds(base, BC)], db1)
        pltpu.sync_copy(dn2_h.at[pl.ds(base, BC)], db2)
        pltpu.sync_copy(inv_h.at[pl.ds(base, BC)], ivb)
        def grp(g, _):
            dd = pl.ds(16 * g, 16)
            ivv = ivb[dd]
            att1 = wb1[dd] / dg1[dd]
            att2 = wb2[dd] / dg2[dd]
            co1 = -2.0 * att1 * db1[dd] * ivv
            co2 = -2.0 * att2 * db2[dd] * ivv
            for e in range(16):
                r = 16 * g + e
                a1, c1 = att1[e], co1[e]
                a2, c2 = att2[e], co2[e]
                for j in range(4):
                    jj = pl.ds(16 * j, 16)
                    rl = relsb[r, jj]
                    rows1[r, jj] = rows1[r, jj] * a1 + rl * c1
                    rows2[r, jj] = rows2[r, jj] * a2 + rl * c2
            return 0
        lax.fori_loop(0, BC // 16, grp, 0, unroll=False)
        pltpu.sync_copy(rows1, acc1.at[dstb.at[0]], add=True)
        pltpu.sync_copy(rows2, acc2.at[dstb.at[0]], add=True)
        return 0
    lax.fori_loop(0, NCHK, chunk, 0, unroll=False)
    plsc.subcore_barrier()

    for acc, g_h in ((acc1, g1_h), (acc2, g2_h)):
        for b in range(RPT // 128):
            r0 = s * RPT + b * 128
            pltpu.sync_copy(acc.at[pl.ds(r0, 128)], fbuf)
            def row(r, _):
                for j in range(4):
                    v = fbuf[r, pl.ds(16 * j, 16)]
                    fbuf[r, pl.ds(16 * j, 16)] = jnp.maximum(v, 0.0)
                return 0
            lax.fori_loop(0, 128, row, 0, unroll=False)
            pltpu.sync_copy(fbuf, g_h.at[pl.ds(r0, 128), c])


_ka_call = functools.partial(
    pl.kernel, _ka_body,
    out_type=[jax.ShapeDtypeStruct((NP, 2, H), _f32),
              jax.ShapeDtypeStruct((NP, 2, H), _f32)],
    mesh=_mesh,
    compiler_params=_params,
    scratch_types=[
        pltpu.VMEM_SHARED((NP, H), _f32),
        pltpu.VMEM_SHARED((NP, H), _f32),
        pltpu.VMEM((1, BC), _i32),
        pltpu.VMEM((1, BC), _i32),
        pltpu.VMEM((BC, H), _f32),
        pltpu.VMEM((BC, H), _f32),
        pltpu.VMEM((BC, H), _f32),
        pltpu.VMEM((BC,), _f32),
        pltpu.VMEM((BC,), _f32),
        pltpu.VMEM((BC,), _f32),
        pltpu.VMEM((BC,), _f32),
        pltpu.VMEM((BC,), _f32),
        pltpu.VMEM((BC,), _f32),
        pltpu.VMEM((BC,), _f32),
        pltpu.VMEM((128, H), _f32),
        pltpu.SemaphoreType.DMA,
    ],
)


_BT = 2048


def _t_logit_body(g1_ref, g2_ref, rels_ref, inv_ref, cf_ref, r2_ref, ak_ref,
                  pl1_ref, pl2_ref, dn1_ref, dn2_ref):
    k1 = ak_ref[1, :]
    rl = rels_ref[...]
    iv = inv_ref[...]
    cf = cf_ref[...]
    r2 = r2_ref[...]
    for g_ref, pl_ref, dn_ref in ((g1_ref, pl1_ref, dn1_ref),
                                  (g2_ref, pl2_ref, dn2_ref)):
        g = g_ref[...]
        a = jnp.sum(g * k1[None, :], axis=1)
        dn = jnp.sum(g * rl, axis=1) * iv
        pl_ref[...] = a - 2.0 * dn * cf + r2
        dn_ref[...] = dn


def _t_logit(g1, g2, rels, inv, cf, r2f, ak3):
    grid = (EP // _BT,)
    bs_r = pl.BlockSpec((_BT, D), lambda i: (i, 0))
    bs_s = pl.BlockSpec((_BT,), lambda i: (i,))
    bs_k = pl.BlockSpec((3, D), lambda i: (0, 0))
    return pl.pallas_call(
        _t_logit_body,
        grid=grid,
        in_specs=[bs_r, bs_r, bs_r, bs_s, bs_s, bs_s, bs_k],
        out_specs=[bs_s, bs_s, bs_s, bs_s],
        out_shape=[jax.ShapeDtypeStruct((EP,), _f32)] * 4,
    )(g1, g2, rels, inv, cf, r2f, ak3)


def _t_svec_body(f1_ref, f2_ref, ak_ref, s1_ref, s2_ref):
    k0 = ak_ref[0, :]
    s1_ref[...] = jnp.sum(f1_ref[...] * k0[None, :], axis=1)
    s2_ref[...] = jnp.sum(f2_ref[...] * k0[None, :], axis=1)


def _t_svec(f1, f2, ak3):
    grid = (NP // _BT,)
    bs_r = pl.BlockSpec((_BT, D), lambda i: (i, 0))
    bs_s = pl.BlockSpec((_BT,), lambda i: (i,))
    bs_k = pl.BlockSpec((3, D), lambda i: (0, 0))
    return pl.pallas_call(
        _t_svec_body,
        grid=grid,
        in_specs=[bs_r, bs_r, bs_k],
        out_specs=[bs_s, bs_s],
        out_shape=[jax.ShapeDtypeStruct((NP,), _f32)] * 2,
    )(f1, f2, ak3)


def _t_rels_body(rels_ref, ak0_ref, ak1_ref,
                 inv_ref, cf0_ref, cf1_ref, r20_ref, r21_ref):
    rl = rels_ref[...]
    iv = 1.0 / jnp.maximum(jnp.sum(jnp.abs(rl), axis=1), 1e-12)
    inv_ref[...] = iv
    cf0_ref[...] = jnp.sum(rl * ak0_ref[1, :][None, :], axis=1) * iv
    cf1_ref[...] = jnp.sum(rl * ak1_ref[1, :][None, :], axis=1) * iv
    r20_ref[...] = jnp.sum(rl * ak0_ref[2, :][None, :], axis=1) * iv
    r21_ref[...] = jnp.sum(rl * ak1_ref[2, :][None, :], axis=1) * iv


def _t_rels(rels, ak0_3, ak1_3):
    grid = (EP // _BT,)
    bs_r = pl.BlockSpec((_BT, D), lambda i: (i, 0))
    bs_s = pl.BlockSpec((_BT,), lambda i: (i,))
    bs_k = pl.BlockSpec((3, D), lambda i: (0, 0))
    return pl.pallas_call(
        _t_rels_body,
        grid=grid,
        in_specs=[bs_r, bs_k, bs_k],
        out_specs=[bs_s] * 5,
        out_shape=[jax.ShapeDtypeStruct((EP,), _f32)] * 5,
    )(rels, ak0_3, ak1_3)


def _pad1(x, v):
    return jnp.concatenate([x, jnp.full((EP - E,), v, x.dtype)])


def _att_pair(dst_p, src_p, F1, F2, ak0, ak1, rels_p, inv_p, cf_p, r2f_p):
    aks = [ak0.reshape(3, D), ak1.reshape(3, D)]
    rels_flat = rels_p.reshape(EP, D)
    outs1, outs2 = [F1], [F2]
    for l in range(2):
        f1f = F1.reshape(2 * NP, H)
        f2f = F2.reshape(2 * NP, H)
        g1, g2 = _kg_call()(src_p, f1f, f2f)
        pl1, pl2, dn1, dn2 = _t_logit(g1.reshape(EP, D), g2.reshape(EP, D),
                                      rels_flat, inv_p, cf_p[l], r2f_p[l],
                                      aks[l])
        s1, s2 = _t_svec(F1.reshape(NP, D), F2.reshape(NP, D), aks[l])
        w1, w2, den1, den2 = _kw_call()(dst_p, s1, s2, pl1, pl2)
        F1, F2 = _ka_call()(dst_p, src_p, f1f, f2f, rels_p,
                            w1, w2, dn1, dn2, inv_p, den1, den2)
        outs1.append(F1)
        outs2.append(F2)
    o1 = jnp.concatenate([o.reshape(NP, D)[:N] for o in outs1], axis=1)
    o2 = jnp.concatenate([o.reshape(NP, D)[:N] for o in outs2], axis=1)
    return o1, o2


def _rels_prep(sp_idx, sp_val, emb, ak0, ak1):
    rels_u = jax.ops.segment_sum(sp_val[:, None] * emb[sp_idx[:, 1]],
                                 sp_idx[:, 0], num_segments=E)
    pad = jnp.zeros((EP - E, D), _f32)
    rels_p = jnp.concatenate([rels_u, pad], axis=0)
    inv, cf0, cf1, r20, r21 = _t_rels(rels_p, ak0.reshape(3, D),
                                      ak1.reshape(3, D))
    return rels_p.reshape(EP, 2, H), inv, [cf0, cf1], [r20, r21]


def kernel(adj_input, r_index, r_val, t_index, ent_matrix, rel_matrix,
           time_matrix, ent_emb_r, ent_emb_t, rel_emb, time_emb,
           ak_e0, ak_e1, ak_t0, ak_t1):
    f_er = _segment_mean_sc(ent_matrix, ent_emb_r)
    f_r = _segment_mean_sc(rel_matrix, rel_emb)
    f_et = _segment_mean_sc(ent_matrix, ent_emb_t)
    f_t = _segment_mean_sc(time_matrix, time_emb)

    dst_p = _pad1(adj_input[:, 0], TRASH)
    src_p = _pad1(adj_input[:, 1], 0)

    rels_e, inv_e, cf_e, r2f_e = _rels_prep(r_index, r_val, rel_emb,
                                            ak_e0, ak_e1)
    rels_t, inv_t, cf_t, r2f_t = _rels_prep(t_index, r_val, time_emb,
                                            ak_t0, ak_t1)

    o1, o2 = _att_pair(dst_p, src_p, f_er, f_r, ak_e0, ak_e1,
                       rels_e, inv_e, cf_e, r2f_e)
    o3, o4 = _att_pair(dst_p, src_p, f_et, f_t, ak_t0, ak_t1,
                       rels_t, inv_t, cf_t, r2f_t)
    output_e_r = jnp.concatenate([o1, o2], axis=-1)
    output_e_t = jnp.concatenate([o3, o4], axis=-1)
    return (output_e_r, output_e_t)


# speedup vs baseline: 2.5195x; 1.1871x over previous
import functools
import jax
import jax.numpy as jnp
from jax import lax
from jax.experimental import pallas as pl
from jax.experimental.pallas import tpu as pltpu
from jax.experimental.pallas import tpu_sc as plsc

N = 10000
E = 320000
D = 128
H = 64
NP = 10240
EP = 327680
TRASH = 10000
RPT = 640
EPT = 20480
BC = 128
NCHK = EPT // BC

_mesh = plsc.VectorSubcoreMesh(core_axis_name="c", subcore_axis_name="s")
_params = pltpu.CompilerParams(use_tc_tiling_on_sc=False)
_f32 = jnp.float32
_i32 = jnp.int32


def _zero_vmem_f32(buf, nrows):
    z = jnp.zeros((16,), _f32)
    def body(r, _):
        for j in range(buf.shape[1] // 16):
            buf[r, pl.ds(16 * j, 16)] = z
        return 0
    lax.fori_loop(0, nrows, body, 0, unroll=False)


def _mean_body(seg_h, col_h, tab_h, outF_h,
               acc, cnt, segb, colb, rows, ones, fbuf, cbuf, sem):
    c = lax.axis_index("c")
    s = lax.axis_index("s")

    _zero_vmem_f32(fbuf, 128)
    z = jnp.zeros((16,), _f32)
    one = jnp.ones((16,), _f32)
    def initones(r, _):
        ones[pl.ds(16 * r, 16)] = one
        cbuf[pl.ds(16 * r, 16)] = z
        return 0
    lax.fori_loop(0, BC // 16, initones, 0, unroll=False)
    for b in range(RPT // 128):
        pltpu.sync_copy(fbuf, acc.at[pl.ds(s * RPT + 128 * b, 128)])
        pltpu.sync_copy(cbuf, cnt.at[pl.ds(s * RPT + 128 * b, 128)])
    plsc.subcore_barrier()

    ebase = s * EPT
    def chunk(k, _):
        base = ebase + k * BC
        pltpu.sync_copy(seg_h.at[pl.ds(base, BC)], segb.at[0])
        pltpu.sync_copy(col_h.at[pl.ds(base, BC)], colb.at[0])
        for j in range(BC // 16):
            v = colb[0, pl.ds(16 * j, 16)]
            colb[0, pl.ds(16 * j, 16)] = v * 2 + c
        pltpu.async_copy(tab_h.at[colb.at[0]], rows, sem).wait()
        pltpu.sync_copy(rows, acc.at[segb.at[0]], add=True)
        pltpu.sync_copy(ones, cnt.at[segb.at[0]], add=True)
        return 0
    lax.fori_loop(0, NCHK, chunk, 0, unroll=False)
    plsc.subcore_barrier()

    for b in range(RPT // 128):
        r0 = s * RPT + b * 128
        pltpu.sync_copy(acc.at[pl.ds(r0, 128)], fbuf)
        pltpu.sync_copy(cnt.at[pl.ds(r0, 128)], cbuf)
        def grp(g, _):
            cntv = cbuf[pl.ds(16 * g, 16)]
            recv = 1.0 / jnp.maximum(cntv, 1.0)
            for r in range(16):
                rec = recv[r]
                for j in range(4):
                    v = fbuf[16 * g + r, pl.ds(16 * j, 16)]
                    fbuf[16 * g + r, pl.ds(16 * j, 16)] = jnp.maximum(v * rec, 0.0)
            return 0
        lax.fori_loop(0, 8, grp, 0, unroll=False)
        pltpu.sync_copy(fbuf, outF_h.at[pl.ds(r0, 128), c])


_mean_call = functools.partial(
    pl.kernel, _mean_body,
    out_type=jax.ShapeDtypeStruct((NP, 2, H), _f32),
    mesh=_mesh,
    compiler_params=_params,
    scratch_types=[
        pltpu.VMEM_SHARED((NP, H), _f32),
        pltpu.VMEM_SHARED((NP,), _f32),
        pltpu.VMEM((1, BC), _i32),
        pltpu.VMEM((1, BC), _i32),
        pltpu.VMEM((BC, H), _f32),
        pltpu.VMEM((BC,), _f32),
        pltpu.VMEM((128, H), _f32),
        pltpu.VMEM((128,), _f32),
        pltpu.SemaphoreType.DMA,
    ],
)


def _segment_mean_sc(idx, table):
    seg = jnp.concatenate([idx[:, 0], jnp.full((EP - E,), TRASH, _i32)])
    col = jnp.concatenate([idx[:, 1], jnp.zeros((EP - E,), _i32)])
    tab2 = table.reshape(-1, 2, H).reshape(-1, H)
    return _mean_call()(seg, col, tab2)


def _kg_body(src_h, f1_h, f2_h, g1_h, g2_h, srcb, rows1, rows2, sem):
    c = lax.axis_index("c")
    s = lax.axis_index("s")

    def chunk(k, _):
        base = s * EPT + k * BC
        pltpu.sync_copy(src_h.at[pl.ds(base, BC)], srcb.at[0])
        for j in range(BC // 16):
            v = srcb[0, pl.ds(16 * j, 16)]
            srcb[0, pl.ds(16 * j, 16)] = v * 2 + c
        cps = [pltpu.async_copy(f1_h.at[srcb.at[0]], rows1, sem),
               pltpu.async_copy(f2_h.at[srcb.at[0]], rows2, sem)]
        for cp in cps:
            cp.wait()
        pltpu.sync_copy(rows1, g1_h.at[pl.ds(base, BC), c])
        pltpu.sync_copy(rows2, g2_h.at[pl.ds(base, BC), c])
        return 0
    lax.fori_loop(0, NCHK, chunk, 0, unroll=False)


_kg_call = functools.partial(
    pl.kernel, _kg_body,
    out_type=[jax.ShapeDtypeStruct((EP, 2, H), _f32),
              jax.ShapeDtypeStruct((EP, 2, H), _f32)],
    mesh=_mesh,
    compiler_params=_params,
    scratch_types=[
        pltpu.VMEM((1, BC), _i32),
        pltpu.VMEM((BC, H), _f32),
        pltpu.VMEM((BC, H), _f32),
        pltpu.SemaphoreType.DMA,
    ],
)


def _kw_body(dst_h, s1_h, s2_h, pl1_h, pl2_h,
             w1_h, w2_h, den1_h, den2_h,
             den_sh, dstb, sgb, plb, wb, sem):
    c = lax.axis_index("c")
    s = lax.axis_index("s")

    z = jnp.zeros((16,), _f32)
    def zrow(r, _):
        wb[pl.ds(16 * r, 16)] = z
        return 0
    lax.fori_loop(0, BC // 16, zrow, 0, unroll=False)
    for b in range(RPT // 128):
        pltpu.sync_copy(wb, den_sh.at[pl.ds(s * RPT + 128 * b, 128)])
    plsc.subcore_barrier()

    def phase1(s_h, pl_h, w_h):
        def chunk(k, _):
            base = s * EPT + k * BC
            pltpu.sync_copy(dst_h.at[pl.ds(base, BC)], dstb.at[0])
            cps = [pltpu.async_copy(s_h.at[dstb.at[0]], sgb, sem),
                   pltpu.async_copy(pl_h.at[pl.ds(base, BC)], plb, sem)]
            for cp in cps:
                cp.wait()
            def grp(g, _):
                dd = pl.ds(16 * g, 16)
                wb[dd] = jnp.exp(sgb[dd] + plb[dd])
                return 0
            lax.fori_loop(0, BC // 16, grp, 0, unroll=False)
            pltpu.sync_copy(wb, w_h.at[pl.ds(base, BC)])
            pltpu.sync_copy(wb, den_sh.at[dstb.at[0]], add=True)
            return 0
        lax.fori_loop(0, NCHK, chunk, 0, unroll=False)

    pl.when(c == 0)(lambda: phase1(s1_h, pl1_h, w1_h))
    pl.when(c == 1)(lambda: phase1(s2_h, pl2_h, w2_h))
    plsc.subcore_barrier()

    @pl.when(s == 0)
    def _():
        pl.when(c == 0)(lambda: pltpu.sync_copy(den_sh, den1_h))
        pl.when(c == 1)(lambda: pltpu.sync_copy(den_sh, den2_h))


_kw_call = functools.partial(
    pl.kernel, _kw_body,
    out_type=[jax.ShapeDtypeStruct((EP,), _f32),
              jax.ShapeDtypeStruct((EP,), _f32),
              jax.ShapeDtypeStruct((NP,), _f32),
              jax.ShapeDtypeStruct((NP,), _f32)],
    mesh=_mesh,
    compiler_params=_params,
    scratch_types=[
        pltpu.VMEM_SHARED((NP,), _f32),
        pltpu.VMEM((1, BC), _i32),
        pltpu.VMEM((BC,), _f32),
        pltpu.VMEM((BC,), _f32),
        pltpu.VMEM((BC,), _f32),
        pltpu.SemaphoreType.DMA,
    ],
)


def _ka_body(dst_h, src_h, f1_h, f2_h, rels_h, w1_h, w2_h, dn1_h, dn2_h,
             inv_h, den1_h, den2_h, g1_h, g2_h,
             acc1, acc2, dstb, srcb, rows1, rows2, relsb,
             wb1, wb2, db1, db2, ivb, dg1, dg2, fbuf, sem):
    c = lax.axis_index("c")
    s = lax.axis_index("s")

    _zero_vmem_f32(fbuf, 128)
    for b in range(RPT // 128):
        pltpu.sync_copy(fbuf, acc1.at[pl.ds(s * RPT + 128 * b, 128)])
        pltpu.sync_copy(fbuf, acc2.at[pl.ds(s * RPT + 128 * b, 128)])
    plsc.subcore_barrier()

    def chunk(k, _):
        base = s * EPT + k * BC
        pltpu.sync_copy(dst_h.at[pl.ds(base, BC)], dstb.at[0])
        pltpu.sync_copy(src_h.at[pl.ds(base, BC)], srcb.at[0])
        for j in range(BC // 16):
            v = srcb[0, pl.ds(16 * j, 16)]
            srcb[0, pl.ds(16 * j, 16)] = v * 2 + c
        cps = [pltpu.async_copy(f1_h.at[srcb.at[0]], rows1, sem),
               pltpu.async_copy(f2_h.at[srcb.at[0]], rows2, sem),
               pltpu.async_copy(den1_h.at[dstb.at[0]], dg1, sem),
               pltpu.async_copy(den2_h.at[dstb.at[0]], dg2, sem),
               pltpu.async_copy(rels_h.at[pl.ds(base, BC), c], relsb, sem),
               pltpu.async_copy(w1_h.at[pl.ds(base, BC)], wb1, sem),
               pltpu.async_copy(w2_h.at[pl.ds(base, BC)], wb2, sem),
               pltpu.async_copy(dn1_h.at[pl.ds(base, BC)], db1, sem),
               pltpu.async_copy(dn2_h.at[pl.ds(base, BC)], db2, sem),
               pltpu.async_copy(inv_h.at[pl.ds(base, BC)], ivb, sem)]
        for cp in cps:
            cp.wait()
        def grp(g, _):
            dd = pl.ds(16 * g, 16)
            ivv = ivb[dd]
            att1 = wb1[dd] / dg1[dd]
            att2 = wb2[dd] / dg2[dd]
            co1 = -2.0 * att1 * db1[dd] * ivv
            co2 = -2.0 * att2 * db2[dd] * ivv
            for e in range(16):
                r = 16 * g + e
                a1, c1 = att1[e], co1[e]
                a2, c2 = att2[e], co2[e]
                for j in range(4):
                    jj = pl.ds(16 * j, 16)
                    rl = relsb[r, jj]
                    rows1[r, jj] = rows1[r, jj] * a1 + rl * c1
                    rows2[r, jj] = rows2[r, jj] * a2 + rl * c2
            return 0
        lax.fori_loop(0, BC // 16, grp, 0, unroll=False)
        pltpu.sync_copy(rows1, acc1.at[dstb.at[0]], add=True)
        pltpu.sync_copy(rows2, acc2.at[dstb.at[0]], add=True)
        return 0
    lax.fori_loop(0, NCHK, chunk, 0, unroll=False)
    plsc.subcore_barrier()

    for acc, g_h in ((acc1, g1_h), (acc2, g2_h)):
        for b in range(RPT // 128):
            r0 = s * RPT + b * 128
            pltpu.sync_copy(acc.at[pl.ds(r0, 128)], fbuf)
            def row(r, _):
                for j in range(4):
                    v = fbuf[r, pl.ds(16 * j, 16)]
                    fbuf[r, pl.ds(16 * j, 16)] = jnp.maximum(v, 0.0)
                return 0
            lax.fori_loop(0, 128, row, 0, unroll=False)
            pltpu.sync_copy(fbuf, g_h.at[pl.ds(r0, 128), c])


_ka_call = functools.partial(
    pl.kernel, _ka_body,
    out_type=[jax.ShapeDtypeStruct((NP, 2, H), _f32),
              jax.ShapeDtypeStruct((NP, 2, H), _f32)],
    mesh=_mesh,
    compiler_params=_params,
    scratch_types=[
        pltpu.VMEM_SHARED((NP, H), _f32),
        pltpu.VMEM_SHARED((NP, H), _f32),
        pltpu.VMEM((1, BC), _i32),
        pltpu.VMEM((1, BC), _i32),
        pltpu.VMEM((BC, H), _f32),
        pltpu.VMEM((BC, H), _f32),
        pltpu.VMEM((BC, H), _f32),
        pltpu.VMEM((BC,), _f32),
        pltpu.VMEM((BC,), _f32),
        pltpu.VMEM((BC,), _f32),
        pltpu.VMEM((BC,), _f32),
        pltpu.VMEM((BC,), _f32),
        pltpu.VMEM((BC,), _f32),
        pltpu.VMEM((BC,), _f32),
        pltpu.VMEM((128, H), _f32),
        pltpu.SemaphoreType.DMA,
    ],
)


_BT = 2048


def _t_logit_body(g1_ref, g2_ref, rels_ref, inv_ref, cf_ref, r2_ref, ak_ref,
                  pl1_ref, pl2_ref, dn1_ref, dn2_ref):
    k1 = ak_ref[1, :]
    rl = rels_ref[...]
    iv = inv_ref[...]
    cf = cf_ref[...]
    r2 = r2_ref[...]
    for g_ref, pl_ref, dn_ref in ((g1_ref, pl1_ref, dn1_ref),
                                  (g2_ref, pl2_ref, dn2_ref)):
        g = g_ref[...]
        a = jnp.sum(g * k1[None, :], axis=1)
        dn = jnp.sum(g * rl, axis=1) * iv
        pl_ref[...] = a - 2.0 * dn * cf + r2
        dn_ref[...] = dn


def _t_logit(g1, g2, rels, inv, cf, r2f, ak3):
    grid = (EP // _BT,)
    bs_r = pl.BlockSpec((_BT, D), lambda i: (i, 0))
    bs_s = pl.BlockSpec((_BT,), lambda i: (i,))
    bs_k = pl.BlockSpec((3, D), lambda i: (0, 0))
    return pl.pallas_call(
        _t_logit_body,
        grid=grid,
        in_specs=[bs_r, bs_r, bs_r, bs_s, bs_s, bs_s, bs_k],
        out_specs=[bs_s, bs_s, bs_s, bs_s],
        out_shape=[jax.ShapeDtypeStruct((EP,), _f32)] * 4,
    )(g1, g2, rels, inv, cf, r2f, ak3)


def _t_svec_body(f1_ref, f2_ref, ak_ref, s1_ref, s2_ref):
    k0 = ak_ref[0, :]
    s1_ref[...] = jnp.sum(f1_ref[...] * k0[None, :], axis=1)
    s2_ref[...] = jnp.sum(f2_ref[...] * k0[None, :], axis=1)


def _t_svec(f1, f2, ak3):
    grid = (NP // _BT,)
    bs_r = pl.BlockSpec((_BT, D), lambda i: (i, 0))
    bs_s = pl.BlockSpec((_BT,), lambda i: (i,))
    bs_k = pl.BlockSpec((3, D), lambda i: (0, 0))
    return pl.pallas_call(
        _t_svec_body,
        grid=grid,
        in_specs=[bs_r, bs_r, bs_k],
        out_specs=[bs_s, bs_s],
        out_shape=[jax.ShapeDtypeStruct((NP,), _f32)] * 2,
    )(f1, f2, ak3)


def _t_rels_body(rels_ref, ak0_ref, ak1_ref,
                 inv_ref, cf0_ref, cf1_ref, r20_ref, r21_ref):
    rl = rels_ref[...]
    iv = 1.0 / jnp.maximum(jnp.sum(jnp.abs(rl), axis=1), 1e-12)
    inv_ref[...] = iv
    cf0_ref[...] = jnp.sum(rl * ak0_ref[1, :][None, :], axis=1) * iv
    cf1_ref[...] = jnp.sum(rl * ak1_ref[1, :][None, :], axis=1) * iv
    r20_ref[...] = jnp.sum(rl * ak0_ref[2, :][None, :], axis=1) * iv
    r21_ref[...] = jnp.sum(rl * ak1_ref[2, :][None, :], axis=1) * iv


def _t_rels(rels, ak0_3, ak1_3):
    grid = (EP // _BT,)
    bs_r = pl.BlockSpec((_BT, D), lambda i: (i, 0))
    bs_s = pl.BlockSpec((_BT,), lambda i: (i,))
    bs_k = pl.BlockSpec((3, D), lambda i: (0, 0))
    return pl.pallas_call(
        _t_rels_body,
        grid=grid,
        in_specs=[bs_r, bs_k, bs_k],
        out_specs=[bs_s] * 5,
        out_shape=[jax.ShapeDtypeStruct((EP,), _f32)] * 5,
    )(rels, ak0_3, ak1_3)


def _pad1(x, v):
    return jnp.concatenate([x, jnp.full((EP - E,), v, x.dtype)])


def _att_pair(dst_p, src_p, F1, F2, ak0, ak1, rels_p, inv_p, cf_p, r2f_p):
    aks = [ak0.reshape(3, D), ak1.reshape(3, D)]
    rels_flat = rels_p.reshape(EP, D)
    outs1, outs2 = [F1], [F2]
    for l in range(2):
        f1f = F1.reshape(2 * NP, H)
        f2f = F2.reshape(2 * NP, H)
        g1, g2 = _kg_call()(src_p, f1f, f2f)
        pl1, pl2, dn1, dn2 = _t_logit(g1.reshape(EP, D), g2.reshape(EP, D),
                                      rels_flat, inv_p, cf_p[l], r2f_p[l],
                                      aks[l])
        s1, s2 = _t_svec(F1.reshape(NP, D), F2.reshape(NP, D), aks[l])
        w1, w2, den1, den2 = _kw_call()(dst_p, s1, s2, pl1, pl2)
        F1, F2 = _ka_call()(dst_p, src_p, f1f, f2f, rels_p,
                            w1, w2, dn1, dn2, inv_p, den1, den2)
        outs1.append(F1)
        outs2.append(F2)
    o1 = jnp.concatenate([o.reshape(NP, D)[:N] for o in outs1], axis=1)
    o2 = jnp.concatenate([o.reshape(NP, D)[:N] for o in outs2], axis=1)
    return o1, o2


def _rels_prep(sp_idx, sp_val, emb, ak0, ak1):
    rels_u = jax.ops.segment_sum(sp_val[:, None] * emb[sp_idx[:, 1]],
                                 sp_idx[:, 0], num_segments=E)
    pad = jnp.zeros((EP - E, D), _f32)
    rels_p = jnp.concatenate([rels_u, pad], axis=0)
    inv, cf0, cf1, r20, r21 = _t_rels(rels_p, ak0.reshape(3, D),
                                      ak1.reshape(3, D))
    return rels_p.reshape(EP, 2, H), inv, [cf0, cf1], [r20, r21]


def kernel(adj_input, r_index, r_val, t_index, ent_matrix, rel_matrix,
           time_matrix, ent_emb_r, ent_emb_t, rel_emb, time_emb,
           ak_e0, ak_e1, ak_t0, ak_t1):
    f_er = _segment_mean_sc(ent_matrix, ent_emb_r)
    f_r = _segment_mean_sc(rel_matrix, rel_emb)
    f_et = _segment_mean_sc(ent_matrix, ent_emb_t)
    f_t = _segment_mean_sc(time_matrix, time_emb)

    dst_p = _pad1(adj_input[:, 0], TRASH)
    src_p = _pad1(adj_input[:, 1], 0)

    rels_e, inv_e, cf_e, r2f_e = _rels_prep(r_index, r_val, rel_emb,
                                            ak_e0, ak_e1)
    rels_t, inv_t, cf_t, r2f_t = _rels_prep(t_index, r_val, time_emb,
                                            ak_t0, ak_t1)

    o1, o2 = _att_pair(dst_p, src_p, f_er, f_r, ak_e0, ak_e1,
                       rels_e, inv_e, cf_e, r2f_e)
    o3, o4 = _att_pair(dst_p, src_p, f_et, f_t, ak_t0, ak_t1,
                       rels_t, inv_t, cf_t, r2f_t)
    output_e_r = jnp.concatenate([o1, o2], axis=-1)
    output_e_t = jnp.concatenate([o3, o4], axis=-1)
    return (output_e_r, output_e_t)


# speedup vs baseline: 2.6182x; 1.0392x over previous
import functools
import jax
import jax.numpy as jnp
from jax import lax
from jax.experimental import pallas as pl
from jax.experimental.pallas import tpu as pltpu
from jax.experimental.pallas import tpu_sc as plsc

N = 10000
E = 320000
D = 128
H = 64
NP = 10240
EP = 327680
TRASH = 10000
RPT = 640
EPT = 20480
BC = 128
NCHK = EPT // BC

_mesh = plsc.VectorSubcoreMesh(core_axis_name="c", subcore_axis_name="s")
_params = pltpu.CompilerParams(use_tc_tiling_on_sc=False)
_f32 = jnp.float32
_i32 = jnp.int32


def _zero_vmem_f32(buf, nrows):
    z = jnp.zeros((16,), _f32)
    def body(r, _):
        for j in range(buf.shape[1] // 16):
            buf[r, pl.ds(16 * j, 16)] = z
        return 0
    lax.fori_loop(0, nrows, body, 0, unroll=False)


def _mean_body(seg_h, col_h, tab_h, outF_h,
               acc, cnt, segb, colb, rows, ones, fbuf, cbuf, sem):
    c = lax.axis_index("c")
    s = lax.axis_index("s")

    _zero_vmem_f32(fbuf, 128)
    z = jnp.zeros((16,), _f32)
    one = jnp.ones((16,), _f32)
    def initones(r, _):
        ones[pl.ds(16 * r, 16)] = one
        cbuf[pl.ds(16 * r, 16)] = z
        return 0
    lax.fori_loop(0, BC // 16, initones, 0, unroll=False)
    for b in range(RPT // 128):
        pltpu.sync_copy(fbuf, acc.at[pl.ds(s * RPT + 128 * b, 128)])
        pltpu.sync_copy(cbuf, cnt.at[pl.ds(s * RPT + 128 * b, 128)])
    plsc.subcore_barrier()

    ebase = s * EPT
    def chunk(k, _):
        base = ebase + k * BC
        pltpu.sync_copy(seg_h.at[pl.ds(base, BC)], segb.at[0])
        pltpu.sync_copy(col_h.at[pl.ds(base, BC)], colb.at[0])
        for j in range(BC // 16):
            v = colb[0, pl.ds(16 * j, 16)]
            colb[0, pl.ds(16 * j, 16)] = v * 2 + c
        pltpu.async_copy(tab_h.at[colb.at[0]], rows, sem).wait()
        pltpu.sync_copy(rows, acc.at[segb.at[0]], add=True)
        pltpu.sync_copy(ones, cnt.at[segb.at[0]], add=True)
        return 0
    lax.fori_loop(0, NCHK, chunk, 0, unroll=False)
    plsc.subcore_barrier()

    for b in range(RPT // 128):
        r0 = s * RPT + b * 128
        pltpu.sync_copy(acc.at[pl.ds(r0, 128)], fbuf)
        pltpu.sync_copy(cnt.at[pl.ds(r0, 128)], cbuf)
        def grp(g, _):
            cntv = cbuf[pl.ds(16 * g, 16)]
            recv = 1.0 / jnp.maximum(cntv, 1.0)
            for r in range(16):
                rec = recv[r]
                for j in range(4):
                    v = fbuf[16 * g + r, pl.ds(16 * j, 16)]
                    fbuf[16 * g + r, pl.ds(16 * j, 16)] = jnp.maximum(v * rec, 0.0)
            return 0
        lax.fori_loop(0, 8, grp, 0, unroll=False)
        pltpu.sync_copy(fbuf, outF_h.at[pl.ds(r0, 128), c])


_mean_call = functools.partial(
    pl.kernel, _mean_body,
    out_type=jax.ShapeDtypeStruct((NP, 2, H), _f32),
    mesh=_mesh,
    compiler_params=_params,
    scratch_types=[
        pltpu.VMEM_SHARED((NP, H), _f32),
        pltpu.VMEM_SHARED((NP,), _f32),
        pltpu.VMEM((1, BC), _i32),
        pltpu.VMEM((1, BC), _i32),
        pltpu.VMEM((BC, H), _f32),
        pltpu.VMEM((BC,), _f32),
        pltpu.VMEM((128, H), _f32),
        pltpu.VMEM((128,), _f32),
        pltpu.SemaphoreType.DMA,
    ],
)


def _segment_mean_sc(idx, table):
    seg = jnp.concatenate([idx[:, 0], jnp.full((EP - E,), TRASH, _i32)])
    col = jnp.concatenate([idx[:, 1], jnp.zeros((EP - E,), _i32)])
    tab2 = table.reshape(-1, 2, H).reshape(-1, H)
    return _mean_call()(seg, col, tab2)


def _kg_body(src_h, f1_h, f2_h, g1_h, g2_h, srcb, rows1, rows2, sem):
    c = lax.axis_index("c")
    s = lax.axis_index("s")

    def chunk(k, _):
        base = s * EPT + k * BC
        pltpu.sync_copy(src_h.at[pl.ds(base, BC)], srcb.at[0])
        for j in range(BC // 16):
            v = srcb[0, pl.ds(16 * j, 16)]
            srcb[0, pl.ds(16 * j, 16)] = v * 2 + c
        cps = [pltpu.async_copy(f1_h.at[srcb.at[0]], rows1, sem),
               pltpu.async_copy(f2_h.at[srcb.at[0]], rows2, sem)]
        for cp in cps:
            cp.wait()
        pltpu.sync_copy(rows1, g1_h.at[pl.ds(base, BC), c])
        pltpu.sync_copy(rows2, g2_h.at[pl.ds(base, BC), c])
        return 0
    lax.fori_loop(0, NCHK, chunk, 0, unroll=False)


_kg_call = functools.partial(
    pl.kernel, _kg_body,
    out_type=[jax.ShapeDtypeStruct((EP, 2, H), _f32),
              jax.ShapeDtypeStruct((EP, 2, H), _f32)],
    mesh=_mesh,
    compiler_params=_params,
    scratch_types=[
        pltpu.VMEM((1, BC), _i32),
        pltpu.VMEM((BC, H), _f32),
        pltpu.VMEM((BC, H), _f32),
        pltpu.SemaphoreType.DMA,
    ],
)


def _kw_body(dst_h, s1_h, s2_h, pl1_h, pl2_h,
             w1_h, w2_h, den1_h, den2_h,
             den_sh, dstb, sgb, plb, wb, sem):
    c = lax.axis_index("c")
    s = lax.axis_index("s")

    z = jnp.zeros((16,), _f32)
    def zrow(r, _):
        wb[pl.ds(16 * r, 16)] = z
        return 0
    lax.fori_loop(0, BC // 16, zrow, 0, unroll=False)
    for b in range(RPT // 128):
        pltpu.sync_copy(wb, den_sh.at[pl.ds(s * RPT + 128 * b, 128)])
    plsc.subcore_barrier()

    def phase1(s_h, pl_h, w_h):
        def chunk(k, _):
            base = s * EPT + k * BC
            pltpu.sync_copy(dst_h.at[pl.ds(base, BC)], dstb.at[0])
            cps = [pltpu.async_copy(s_h.at[dstb.at[0]], sgb, sem),
                   pltpu.async_copy(pl_h.at[pl.ds(base, BC)], plb, sem)]
            for cp in cps:
                cp.wait()
            def grp(g, _):
                dd = pl.ds(16 * g, 16)
                wb[dd] = jnp.exp(sgb[dd] + plb[dd])
                return 0
            lax.fori_loop(0, BC // 16, grp, 0, unroll=False)
            pltpu.sync_copy(wb, w_h.at[pl.ds(base, BC)])
            pltpu.sync_copy(wb, den_sh.at[dstb.at[0]], add=True)
            return 0
        lax.fori_loop(0, NCHK, chunk, 0, unroll=False)

    pl.when(c == 0)(lambda: phase1(s1_h, pl1_h, w1_h))
    pl.when(c == 1)(lambda: phase1(s2_h, pl2_h, w2_h))
    plsc.subcore_barrier()

    @pl.when(s == 0)
    def _():
        pl.when(c == 0)(lambda: pltpu.sync_copy(den_sh, den1_h))
        pl.when(c == 1)(lambda: pltpu.sync_copy(den_sh, den2_h))


_kw_call = functools.partial(
    pl.kernel, _kw_body,
    out_type=[jax.ShapeDtypeStruct((EP,), _f32),
              jax.ShapeDtypeStruct((EP,), _f32),
              jax.ShapeDtypeStruct((NP,), _f32),
              jax.ShapeDtypeStruct((NP,), _f32)],
    mesh=_mesh,
    compiler_params=_params,
    scratch_types=[
        pltpu.VMEM_SHARED((NP,), _f32),
        pltpu.VMEM((1, BC), _i32),
        pltpu.VMEM((BC,), _f32),
        pltpu.VMEM((BC,), _f32),
        pltpu.VMEM((BC,), _f32),
        pltpu.SemaphoreType.DMA,
    ],
)


def _ka_body(dst_h, g1s_h, g2s_h, rels_h, w1_h, w2_h, dn1_h, dn2_h,
             inv_h, den1_h, den2_h, g1_h, g2_h,
             acc1, acc2, dstb, rows1, rows2, relsb,
             wb1, wb2, db1, db2, ivb, dg1, dg2, fbuf, sem):
    c = lax.axis_index("c")
    s = lax.axis_index("s")

    _zero_vmem_f32(fbuf, 128)
    for b in range(RPT // 128):
        pltpu.sync_copy(fbuf, acc1.at[pl.ds(s * RPT + 128 * b, 128)])
        pltpu.sync_copy(fbuf, acc2.at[pl.ds(s * RPT + 128 * b, 128)])
    plsc.subcore_barrier()

    def chunk(k, _):
        base = s * EPT + k * BC
        pltpu.sync_copy(dst_h.at[pl.ds(base, BC)], dstb.at[0])
        cps = [pltpu.async_copy(g1s_h.at[pl.ds(base, BC), c], rows1, sem),
               pltpu.async_copy(g2s_h.at[pl.ds(base, BC), c], rows2, sem),
               pltpu.async_copy(den1_h.at[dstb.at[0]], dg1, sem),
               pltpu.async_copy(den2_h.at[dstb.at[0]], dg2, sem),
               pltpu.async_copy(rels_h.at[pl.ds(base, BC), c], relsb, sem),
               pltpu.async_copy(w1_h.at[pl.ds(base, BC)], wb1, sem),
               pltpu.async_copy(w2_h.at[pl.ds(base, BC)], wb2, sem),
               pltpu.async_copy(dn1_h.at[pl.ds(base, BC)], db1, sem),
               pltpu.async_copy(dn2_h.at[pl.ds(base, BC)], db2, sem),
               pltpu.async_copy(inv_h.at[pl.ds(base, BC)], ivb, sem)]
        for cp in cps:
            cp.wait()
        def grp(g, _):
            dd = pl.ds(16 * g, 16)
            ivv = ivb[dd]
            att1 = wb1[dd] / dg1[dd]
            att2 = wb2[dd] / dg2[dd]
            co1 = -2.0 * att1 * db1[dd] * ivv
            co2 = -2.0 * att2 * db2[dd] * ivv
            for e in range(16):
                r = 16 * g + e
                a1, c1 = att1[e], co1[e]
                a2, c2 = att2[e], co2[e]
                for j in range(4):
                    jj = pl.ds(16 * j, 16)
                    rl = relsb[r, jj]
                    rows1[r, jj] = rows1[r, jj] * a1 + rl * c1
                    rows2[r, jj] = rows2[r, jj] * a2 + rl * c2
            return 0
        lax.fori_loop(0, BC // 16, grp, 0, unroll=False)
        pltpu.sync_copy(rows1, acc1.at[dstb.at[0]], add=True)
        pltpu.sync_copy(rows2, acc2.at[dstb.at[0]], add=True)
        return 0
    lax.fori_loop(0, NCHK, chunk, 0, unroll=False)
    plsc.subcore_barrier()

    for acc, g_h in ((acc1, g1_h), (acc2, g2_h)):
        for b in range(RPT // 128):
            r0 = s * RPT + b * 128
            pltpu.sync_copy(acc.at[pl.ds(r0, 128)], fbuf)
            def row(r, _):
                for j in range(4):
                    v = fbuf[r, pl.ds(16 * j, 16)]
                    fbuf[r, pl.ds(16 * j, 16)] = jnp.maximum(v, 0.0)
                return 0
            lax.fori_loop(0, 128, row, 0, unroll=False)
            pltpu.sync_copy(fbuf, g_h.at[pl.ds(r0, 128), c])


_ka_call = functools.partial(
    pl.kernel, _ka_body,
    out_type=[jax.ShapeDtypeStruct((NP, 2, H), _f32),
              jax.ShapeDtypeStruct((NP, 2, H), _f32)],
    mesh=_mesh,
    compiler_params=_params,
    scratch_types=[
        pltpu.VMEM_SHARED((NP, H), _f32),
        pltpu.VMEM_SHARED((NP, H), _f32),
        pltpu.VMEM((1, BC), _i32),
        pltpu.VMEM((BC, H), _f32),
        pltpu.VMEM((BC, H), _f32),
        pltpu.VMEM((BC, H), _f32),
        pltpu.VMEM((BC,), _f32),
        pltpu.VMEM((BC,), _f32),
        pltpu.VMEM((BC,), _f32),
        pltpu.VMEM((BC,), _f32),
        pltpu.VMEM((BC,), _f32),
        pltpu.VMEM((BC,), _f32),
        pltpu.VMEM((BC,), _f32),
        pltpu.VMEM((128, H), _f32),
        pltpu.SemaphoreType.DMA,
    ],
)


_BT = 2048


def _t_logit_body(g1_ref, g2_ref, rels_ref, inv_ref, cf_ref, r2_ref, ak_ref,
                  pl1_ref, pl2_ref, dn1_ref, dn2_ref):
    k1 = ak_ref[1, :]
    rl = rels_ref[...]
    iv = inv_ref[...]
    cf = cf_ref[...]
    r2 = r2_ref[...]
    for g_ref, pl_ref, dn_ref in ((g1_ref, pl1_ref, dn1_ref),
                                  (g2_ref, pl2_ref, dn2_ref)):
        g = g_ref[...]
        a = jnp.sum(g * k1[None, :], axis=1)
        dn = jnp.sum(g * rl, axis=1) * iv
        pl_ref[...] = a - 2.0 * dn * cf + r2
        dn_ref[...] = dn


def _t_logit(g1, g2, rels, inv, cf, r2f, ak3):
    grid = (EP // _BT,)
    bs_r = pl.BlockSpec((_BT, D), lambda i: (i, 0))
    bs_s = pl.BlockSpec((_BT,), lambda i: (i,))
    bs_k = pl.BlockSpec((3, D), lambda i: (0, 0))
    return pl.pallas_call(
        _t_logit_body,
        grid=grid,
        in_specs=[bs_r, bs_r, bs_r, bs_s, bs_s, bs_s, bs_k],
        out_specs=[bs_s, bs_s, bs_s, bs_s],
        out_shape=[jax.ShapeDtypeStruct((EP,), _f32)] * 4,
    )(g1, g2, rels, inv, cf, r2f, ak3)


def _t_svec_body(f1_ref, f2_ref, ak_ref, s1_ref, s2_ref):
    k0 = ak_ref[0, :]
    s1_ref[...] = jnp.sum(f1_ref[...] * k0[None, :], axis=1)
    s2_ref[...] = jnp.sum(f2_ref[...] * k0[None, :], axis=1)


def _t_svec(f1, f2, ak3):
    grid = (NP // _BT,)
    bs_r = pl.BlockSpec((_BT, D), lambda i: (i, 0))
    bs_s = pl.BlockSpec((_BT,), lambda i: (i,))
    bs_k = pl.BlockSpec((3, D), lambda i: (0, 0))
    return pl.pallas_call(
        _t_svec_body,
        grid=grid,
        in_specs=[bs_r, bs_r, bs_k],
        out_specs=[bs_s, bs_s],
        out_shape=[jax.ShapeDtypeStruct((NP,), _f32)] * 2,
    )(f1, f2, ak3)


def _t_rels_body(rels_ref, ak0_ref, ak1_ref,
                 inv_ref, cf0_ref, cf1_ref, r20_ref, r21_ref):
    rl = rels_ref[...]
    iv = 1.0 / jnp.maximum(jnp.sum(jnp.abs(rl), axis=1), 1e-12)
    inv_ref[...] = iv
    cf0_ref[...] = jnp.sum(rl * ak0_ref[1, :][None, :], axis=1) * iv
    cf1_ref[...] = jnp.sum(rl * ak1_ref[1, :][None, :], axis=1) * iv
    r20_ref[...] = jnp.sum(rl * ak0_ref[2, :][None, :], axis=1) * iv
    r21_ref[...] = jnp.sum(rl * ak1_ref[2, :][None, :], axis=1) * iv


def _t_rels(rels, ak0_3, ak1_3):
    grid = (EP // _BT,)
    bs_r = pl.BlockSpec((_BT, D), lambda i: (i, 0))
    bs_s = pl.BlockSpec((_BT,), lambda i: (i,))
    bs_k = pl.BlockSpec((3, D), lambda i: (0, 0))
    return pl.pallas_call(
        _t_rels_body,
        grid=grid,
        in_specs=[bs_r, bs_k, bs_k],
        out_specs=[bs_s] * 5,
        out_shape=[jax.ShapeDtypeStruct((EP,), _f32)] * 5,
    )(rels, ak0_3, ak1_3)


def _pad1(x, v):
    return jnp.concatenate([x, jnp.full((EP - E,), v, x.dtype)])


def _att_pair(dst_p, src_p, F1, F2, ak0, ak1, rels_p, inv_p, cf_p, r2f_p):
    aks = [ak0.reshape(3, D), ak1.reshape(3, D)]
    rels_flat = rels_p.reshape(EP, D)
    outs1, outs2 = [F1], [F2]
    for l in range(2):
        f1f = F1.reshape(2 * NP, H)
        f2f = F2.reshape(2 * NP, H)
        g1, g2 = _kg_call()(src_p, f1f, f2f)
        pl1, pl2, dn1, dn2 = _t_logit(g1.reshape(EP, D), g2.reshape(EP, D),
                                      rels_flat, inv_p, cf_p[l], r2f_p[l],
                                      aks[l])
        s1, s2 = _t_svec(F1.reshape(NP, D), F2.reshape(NP, D), aks[l])
        w1, w2, den1, den2 = _kw_call()(dst_p, s1, s2, pl1, pl2)
        F1, F2 = _ka_call()(dst_p, g1, g2, rels_p,
                            w1, w2, dn1, dn2, inv_p, den1, den2)
        outs1.append(F1)
        outs2.append(F2)
    o1 = jnp.concatenate([o.reshape(NP, D)[:N] for o in outs1], axis=1)
    o2 = jnp.concatenate([o.reshape(NP, D)[:N] for o in outs2], axis=1)
    return o1, o2


def _rels_prep(sp_idx, sp_val, emb, ak0, ak1):
    rels_u = jax.ops.segment_sum(sp_val[:, None] * emb[sp_idx[:, 1]],
                                 sp_idx[:, 0], num_segments=E)
    pad = jnp.zeros((EP - E, D), _f32)
    rels_p = jnp.concatenate([rels_u, pad], axis=0)
    inv, cf0, cf1, r20, r21 = _t_rels(rels_p, ak0.reshape(3, D),
                                      ak1.reshape(3, D))
    return rels_p.reshape(EP, 2, H), inv, [cf0, cf1], [r20, r21]


def kernel(adj_input, r_index, r_val, t_index, ent_matrix, rel_matrix,
           time_matrix, ent_emb_r, ent_emb_t, rel_emb, time_emb,
           ak_e0, ak_e1, ak_t0, ak_t1):
    f_er = _segment_mean_sc(ent_matrix, ent_emb_r)
    f_r = _segment_mean_sc(rel_matrix, rel_emb)
    f_et = _segment_mean_sc(ent_matrix, ent_emb_t)
    f_t = _segment_mean_sc(time_matrix, time_emb)

    dst_p = _pad1(adj_input[:, 0], TRASH)
    src_p = _pad1(adj_input[:, 1], 0)

    rels_e, inv_e, cf_e, r2f_e = _rels_prep(r_index, r_val, rel_emb,
                                            ak_e0, ak_e1)
    rels_t, inv_t, cf_t, r2f_t = _rels_prep(t_index, r_val, time_emb,
                                            ak_t0, ak_t1)

    o1, o2 = _att_pair(dst_p, src_p, f_er, f_r, ak_e0, ak_e1,
                       rels_e, inv_e, cf_e, r2f_e)
    o3, o4 = _att_pair(dst_p, src_p, f_et, f_t, ak_t0, ak_t1,
                       rels_t, inv_t, cf_t, r2f_t)
    output_e_r = jnp.concatenate([o1, o2], axis=-1)
    output_e_t = jnp.concatenate([o3, o4], axis=-1)
    return (output_e_r, output_e_t)
